# Initial kernel scaffold; baseline (speedup 1.0000x reference)
import jax
import jax.numpy as jnp
from jax.experimental import pallas as pl


def kernel(x, hidden_state, edge_index, params):
    raise NotImplementedError("write your pallas kernel here")


# speedup vs baseline: 2.7130x; 2.7130x over previous
import jax
import jax.numpy as jnp
from jax.experimental import pallas as pl

_EDGE_LIST = [(0, 9), (0, 10), (0, 5), (9, 7), (9, 8), (9, 4), (2, 10), (2, 5),
              (7, 9), (7, 5), (7, 4), (8, 9), (8, 4), (6, 0), (6, 2), (6, 5),
              (6, 9), (3, 10), (3, 5), (10, 5), (1, 0), (1, 3)]
_N = 11
_SRC = tuple(e[0] for e in _EDGE_LIST) + tuple(range(_N))
_DST = tuple(e[1] for e in _EDGE_LIST) + tuple(range(_N))
_E = len(_SRC)
_IN_EDGES = tuple(tuple(k for k in range(_E) if _DST[k] == i) for i in range(_N))

_H, _C = 2, 32
_HC = _H * _C
_HID = 32


def _elu(v):
    return jnp.where(v > 0, v, jnp.exp(jnp.minimum(v, 0.0)) - 1.0)


def _gat(x2, Wl, bl, Wr, br, Ab, bias, bb):
    xl = (jnp.dot(x2, Wl, preferred_element_type=jnp.float32) + bl).reshape(_N, bb, _HC)
    xr = (jnp.dot(x2, Wr, preferred_element_type=jnp.float32) + br).reshape(_N, bb, _HC)
    xj = jnp.stack([xl[s] for s in _SRC])
    xi = jnp.stack([xr[d] for d in _DST])
    e = jax.nn.leaky_relu(xj + xi, negative_slope=0.2)
    lb = jnp.dot(e.reshape(_E * bb, _HC), Ab,
                 preferred_element_type=jnp.float32).reshape(_E, bb, _HC)
    ex = jnp.exp(lb)
    m = ex * xj
    rows = []
    for i in range(_N):
        ks = _IN_EDGES[i]
        sm = ex[ks[0]]
        acc = m[ks[0]]
        for k in ks[1:]:
            sm = sm + ex[k]
            acc = acc + m[k]
        rows.append(acc / (sm + 1e-16))
    return jnp.stack(rows) + bias


def _body(x_ref, h0_ref, Wl1, bl1, Wr1, br1, Ab1, b1, Wl2, bl2, Wr2, br2,
          Ab2, b2, WihT, bih, WhhT, bhh, Hw1, Hb1, Hw2, Hb2,
          out_ref, hnew_ref):
    bb = x_ref.shape[1]
    x2 = x_ref[...].reshape(_N * bb, x_ref.shape[2])
    h = _gat(x2, Wl1[...], bl1[...], Wr1[...], br1[...], Ab1[...], b1[...], bb)
    h = _elu(h)
    h = _gat(h.reshape(_N * bb, _HC), Wl2[...], bl2[...], Wr2[...], br2[...],
             Ab2[...], b2[...], bb)
    h = _elu(h)

    gx2 = h.reshape(_N * bb, _HC)
    h02 = h0_ref[...].reshape(_N * bb, _HID)
    gi = jnp.dot(gx2, WihT[...], preferred_element_type=jnp.float32) + bih[...]
    gh = jnp.dot(h02, WhhT[...], preferred_element_type=jnp.float32) + bhh[...]
    rz = jax.nn.sigmoid(gi[:, :2 * _HID] + gh[:, :2 * _HID])
    r = rz[:, :_HID]
    z = rz[:, _HID:]
    n = jnp.tanh(gi[:, 2 * _HID:] + r * gh[:, 2 * _HID:])
    hnew2 = (1.0 - z) * n + z * h02
    hnew_ref[...] = hnew2.reshape(_N, bb, _HID)

    t = hnew2.reshape(_N, bb, _HID)
    outs = []
    for i in range(_N):
        h1 = jax.nn.relu(jnp.dot(t[i], Hw1[i], preferred_element_type=jnp.float32)
                         + Hb1[i])
        outs.append(jnp.dot(h1, Hw2[i], preferred_element_type=jnp.float32) + Hb2[i])
    o = jnp.stack(outs)
    ot = jnp.tanh(o)
    osig = jax.nn.sigmoid(o)
    lane = jax.lax.broadcasted_iota(jnp.int32, o.shape, 2)
    out_ref[...] = jnp.where(lane == 2, osig,
                             jnp.where(lane == 0, ot * 0.3, ot * 0.2))


def _att_mat(att):
    z = jnp.zeros((_C, _C), jnp.float32)
    blocks = []
    for h in range(_H):
        row = [z] * _H
        row[h] = jnp.broadcast_to(att[h][:, None], (_C, _C))
        blocks.append(jnp.concatenate(row, axis=1))
    return jnp.concatenate(blocks, axis=0)


def kernel(x, hidden_state, edge_index, params):
    B, N, D = x.shape
    p = params
    bb = 256
    x_t = x.transpose(1, 0, 2)
    h0_t = hidden_state[0].reshape(B, N, _HID).transpose(1, 0, 2)

    r2 = lambda v: v.reshape(1, -1)
    weights = [
        p['Wl1'], r2(p['bl1']), p['Wr1'], r2(p['br1']), _att_mat(p['att1']), r2(p['bias1']),
        p['Wl2'], r2(p['bl2']), p['Wr2'], r2(p['br2']), _att_mat(p['att2']), r2(p['bias2']),
        p['Wih'].T, r2(p['bih']), p['Whh'].T, r2(p['bhh']),
        p['Hw1'], p['Hb1'].reshape(N, 1, 16), p['Hw2'], p['Hb2'].reshape(N, 1, 3),
    ]

    grid = (B // bb,)
    batch_spec = lambda f: pl.BlockSpec((N, bb, f), lambda i: (0, i, 0))
    w_specs = [pl.BlockSpec(w.shape, (lambda nd: (lambda i: (0,) * nd))(w.ndim))
               for w in weights]

    out_t, hnew_t = pl.pallas_call(
        _body,
        grid=grid,
        in_specs=[batch_spec(D), batch_spec(_HID)] + w_specs,
        out_specs=[batch_spec(3), batch_spec(_HID)],
        out_shape=[jax.ShapeDtypeStruct((N, B, 3), jnp.float32),
                   jax.ShapeDtypeStruct((N, B, _HID), jnp.float32)],
    )(x_t, h0_t, *weights)

    out = out_t.transpose(1, 0, 2)
    hnew = hnew_t.transpose(1, 0, 2).reshape(1, B * N, _HID)
    return out, hnew


# speedup vs baseline: 2.7855x; 1.0267x over previous
import jax
import jax.numpy as jnp
from jax.experimental import pallas as pl

_EDGE_LIST = [(0, 9), (0, 10), (0, 5), (9, 7), (9, 8), (9, 4), (2, 10), (2, 5),
              (7, 9), (7, 5), (7, 4), (8, 9), (8, 4), (6, 0), (6, 2), (6, 5),
              (6, 9), (3, 10), (3, 5), (10, 5), (1, 0), (1, 3)]
_N = 11
_SRC = tuple(e[0] for e in _EDGE_LIST) + tuple(range(_N))
_DST = tuple(e[1] for e in _EDGE_LIST) + tuple(range(_N))
_E = len(_SRC)
_IN_EDGES = tuple(tuple(k for k in range(_E) if _DST[k] == i) for i in range(_N))

_H, _C = 2, 32
_HC = _H * _C
_HID = 32


def _elu(v):
    return jnp.where(v > 0, v, jnp.exp(jnp.minimum(v, 0.0)) - 1.0)


def _gat(x2, Wl, bl, Wr, br, Ab, bias, bb):
    xl = (jnp.dot(x2, Wl, preferred_element_type=jnp.float32) + bl).reshape(_N, bb, _HC)
    xr = (jnp.dot(x2, Wr, preferred_element_type=jnp.float32) + br).reshape(_N, bb, _HC)
    xj = jnp.stack([xl[s] for s in _SRC])
    xi = jnp.stack([xr[d] for d in _DST])
    e = jax.nn.leaky_relu(xj + xi, negative_slope=0.2)
    lb = jnp.dot(e.reshape(_E * bb, _HC), Ab,
                 preferred_element_type=jnp.float32).reshape(_E, bb, _HC)
    ex = jnp.exp(lb)
    m = ex * xj
    rows = []
    for i in range(_N):
        ks = _IN_EDGES[i]
        sm = ex[ks[0]]
        acc = m[ks[0]]
        for k in ks[1:]:
            sm = sm + ex[k]
            acc = acc + m[k]
        rows.append(acc / (sm + 1e-16))
    return jnp.stack(rows) + bias


def _body(x_ref, h0_ref, Wl1, bl1, Wr1, br1, Ab1, b1, Wl2, bl2, Wr2, br2,
          Ab2, b2, WihT, bih, WhhT, bhh, Hw1, Hb1, Hw2, Hb2,
          out_ref, hnew_ref):
    bb = x_ref.shape[1]
    x2 = x_ref[...].reshape(_N * bb, x_ref.shape[2])
    h = _gat(x2, Wl1[...], bl1[...], Wr1[...], br1[...], Ab1[...], b1[...], bb)
    h = _elu(h)
    h = _gat(h.reshape(_N * bb, _HC), Wl2[...], bl2[...], Wr2[...], br2[...],
             Ab2[...], b2[...], bb)
    h = _elu(h)

    gx2 = h.reshape(_N * bb, _HC)
    h02 = h0_ref[...].reshape(_N * bb, _HID)
    gi = jnp.dot(gx2, WihT[...], preferred_element_type=jnp.float32) + bih[...]
    gh = jnp.dot(h02, WhhT[...], preferred_element_type=jnp.float32) + bhh[...]
    rz = jax.nn.sigmoid(gi[:, :2 * _HID] + gh[:, :2 * _HID])
    r = rz[:, :_HID]
    z = rz[:, _HID:]
    n = jnp.tanh(gi[:, 2 * _HID:] + r * gh[:, 2 * _HID:])
    hnew2 = (1.0 - z) * n + z * h02
    hnew_ref[...] = hnew2.reshape(_N, bb, _HID)

    t = hnew2.reshape(_N, bb, _HID)
    outs = []
    for i in range(_N):
        h1 = jax.nn.relu(jnp.dot(t[i], Hw1[i], preferred_element_type=jnp.float32)
                         + Hb1[i])
        outs.append(jnp.dot(h1, Hw2[i], preferred_element_type=jnp.float32) + Hb2[i])
    o = jnp.stack(outs)
    ot = jnp.tanh(o)
    osig = jax.nn.sigmoid(o)
    lane = jax.lax.broadcasted_iota(jnp.int32, o.shape, 2)
    out_ref[...] = jnp.where(lane == 2, osig,
                             jnp.where(lane == 0, ot * 0.3, ot * 0.2))


def _att_mat(att):
    z = jnp.zeros((_C, _C), jnp.float32)
    blocks = []
    for h in range(_H):
        row = [z] * _H
        row[h] = jnp.broadcast_to(att[h][:, None], (_C, _C))
        blocks.append(jnp.concatenate(row, axis=1))
    return jnp.concatenate(blocks, axis=0)


def kernel(x, hidden_state, edge_index, params):
    B, N, D = x.shape
    p = params
    bb = 512
    x_t = x.transpose(1, 0, 2)
    h0_t = hidden_state[0].reshape(B, N, _HID).transpose(1, 0, 2)

    r2 = lambda v: v.reshape(1, -1)
    weights = [
        p['Wl1'], r2(p['bl1']), p['Wr1'], r2(p['br1']), _att_mat(p['att1']), r2(p['bias1']),
        p['Wl2'], r2(p['bl2']), p['Wr2'], r2(p['br2']), _att_mat(p['att2']), r2(p['bias2']),
        p['Wih'].T, r2(p['bih']), p['Whh'].T, r2(p['bhh']),
        p['Hw1'], p['Hb1'].reshape(N, 1, 16), p['Hw2'], p['Hb2'].reshape(N, 1, 3),
    ]

    grid = (B // bb,)
    batch_spec = lambda f: pl.BlockSpec((N, bb, f), lambda i: (0, i, 0))
    w_specs = [pl.BlockSpec(w.shape, (lambda nd: (lambda i: (0,) * nd))(w.ndim))
               for w in weights]

    out_t, hnew_t = pl.pallas_call(
        _body,
        grid=grid,
        in_specs=[batch_spec(D), batch_spec(_HID)] + w_specs,
        out_specs=[batch_spec(3), batch_spec(_HID)],
        out_shape=[jax.ShapeDtypeStruct((N, B, 3), jnp.float32),
                   jax.ShapeDtypeStruct((N, B, _HID), jnp.float32)],
    )(x_t, h0_t, *weights)

    out = out_t.transpose(1, 0, 2)
    hnew = hnew_t.transpose(1, 0, 2).reshape(1, B * N, _HID)
    return out, hnew


# speedup vs baseline: 3.2802x; 1.1776x over previous
import jax
import jax.numpy as jnp
from jax.experimental import pallas as pl

_EDGE_LIST = [(0, 9), (0, 10), (0, 5), (9, 7), (9, 8), (9, 4), (2, 10), (2, 5),
              (7, 9), (7, 5), (7, 4), (8, 9), (8, 4), (6, 0), (6, 2), (6, 5),
              (6, 9), (3, 10), (3, 5), (10, 5), (1, 0), (1, 3)]
_N = 11
_SRC = tuple(e[0] for e in _EDGE_LIST) + tuple(range(_N))
_DST = tuple(e[1] for e in _EDGE_LIST) + tuple(range(_N))
_E = len(_SRC)
_IN_EDGES = tuple(tuple(k for k in range(_E) if _DST[k] == i) for i in range(_N))

_H, _C = 2, 32
_HC = _H * _C
_HID = 32


def _elu(v):
    return jnp.where(v > 0, v, jnp.exp(jnp.minimum(v, 0.0)) - 1.0)


def _gat(x2, Wl, bl, Wr, br, Ab, bias, bb):
    xl = (jnp.dot(x2, Wl, preferred_element_type=jnp.float32) + bl).reshape(_N, bb, _HC)
    xr = (jnp.dot(x2, Wr, preferred_element_type=jnp.float32) + br).reshape(_N, bb, _HC)
    xj = jnp.stack([xl[s] for s in _SRC])
    xi = jnp.stack([xr[d] for d in _DST])
    e = jax.nn.leaky_relu(xj + xi, negative_slope=0.2)
    lb = jnp.dot(e.reshape(_E * bb, _HC), Ab,
                 preferred_element_type=jnp.float32).reshape(_E, bb, _HC)
    ex = jnp.exp(lb)
    m = ex * xj
    rows = []
    for i in range(_N):
        ks = _IN_EDGES[i]
        sm = ex[ks[0]]
        acc = m[ks[0]]
        for k in ks[1:]:
            sm = sm + ex[k]
            acc = acc + m[k]
        rows.append(acc / (sm + 1e-16))
    return jnp.stack(rows) + bias


def _body(x_ref, h0_ref, Wl1, bl1, Wr1, br1, Ab1, b1, Wl2, bl2, Wr2, br2,
          Ab2, b2, WihT, bih, WhhT, bhh, Hw1, Hb1, Hw2, Hb2,
          out_ref, hnew_ref):
    bb = x_ref.shape[0]
    d = x_ref.shape[1] // _N
    xw = x_ref[...]
    x2 = jnp.concatenate([xw[:, n * d:(n + 1) * d] for n in range(_N)], axis=0)
    h = _gat(x2, Wl1[...], bl1[...], Wr1[...], br1[...], Ab1[...], b1[...], bb)
    h = _elu(h)
    h = _gat(h.reshape(_N * bb, _HC), Wl2[...], bl2[...], Wr2[...], br2[...],
             Ab2[...], b2[...], bb)
    h = _elu(h)

    gx2 = h.reshape(_N * bb, _HC)
    h0w = h0_ref[...]
    h02 = jnp.concatenate(
        [h0w[:, n * _HID:(n + 1) * _HID] for n in range(_N)], axis=0)
    gi = jnp.dot(gx2, WihT[...], preferred_element_type=jnp.float32) + bih[...]
    gh = jnp.dot(h02, WhhT[...], preferred_element_type=jnp.float32) + bhh[...]
    rz = jax.nn.sigmoid(gi[:, :2 * _HID] + gh[:, :2 * _HID])
    r = rz[:, :_HID]
    z = rz[:, _HID:]
    n = jnp.tanh(gi[:, 2 * _HID:] + r * gh[:, 2 * _HID:])
    hnew2 = (1.0 - z) * n + z * h02
    t = hnew2.reshape(_N, bb, _HID)
    hnew_ref[...] = jnp.concatenate([t[i] for i in range(_N)], axis=1)
    outs = []
    for i in range(_N):
        h1 = jax.nn.relu(jnp.dot(t[i], Hw1[i], preferred_element_type=jnp.float32)
                         + Hb1[i])
        outs.append(jnp.dot(h1, Hw2[i], preferred_element_type=jnp.float32) + Hb2[i])
    o = jnp.concatenate(outs, axis=1)
    ot = jnp.tanh(o)
    osig = jax.nn.sigmoid(o)
    lane = jax.lax.broadcasted_iota(jnp.int32, o.shape, 1) % 3
    out_ref[...] = jnp.where(lane == 2, osig,
                             jnp.where(lane == 0, ot * 0.3, ot * 0.2))


def _att_mat(att):
    z = jnp.zeros((_C, _C), jnp.float32)
    blocks = []
    for h in range(_H):
        row = [z] * _H
        row[h] = jnp.broadcast_to(att[h][:, None], (_C, _C))
        blocks.append(jnp.concatenate(row, axis=1))
    return jnp.concatenate(blocks, axis=0)


def kernel(x, hidden_state, edge_index, params):
    B, N, D = x.shape
    p = params
    bb = 512
    xw = x.reshape(B, N * D)
    h0w = hidden_state.reshape(B, N * _HID)

    r2 = lambda v: v.reshape(1, -1)
    weights = [
        p['Wl1'], r2(p['bl1']), p['Wr1'], r2(p['br1']), _att_mat(p['att1']), r2(p['bias1']),
        p['Wl2'], r2(p['bl2']), p['Wr2'], r2(p['br2']), _att_mat(p['att2']), r2(p['bias2']),
        p['Wih'].T, r2(p['bih']), p['Whh'].T, r2(p['bhh']),
        p['Hw1'], p['Hb1'].reshape(N, 1, 16), p['Hw2'], p['Hb2'].reshape(N, 1, 3),
    ]

    grid = (B // bb,)
    batch_spec = lambda f: pl.BlockSpec((bb, f), lambda i: (i, 0))
    w_specs = [pl.BlockSpec(w.shape, (lambda nd: (lambda i: (0,) * nd))(w.ndim))
               for w in weights]

    out_w, hnew_w = pl.pallas_call(
        _body,
        grid=grid,
        in_specs=[batch_spec(N * D), batch_spec(N * _HID)] + w_specs,
        out_specs=[batch_spec(N * 3), batch_spec(N * _HID)],
        out_shape=[jax.ShapeDtypeStruct((B, N * 3), jnp.float32),
                   jax.ShapeDtypeStruct((B, N * _HID), jnp.float32)],
    )(xw, h0w, *weights)

    out = out_w.reshape(B, N, 3)
    hnew = hnew_w.reshape(1, B * N, _HID)
    return out, hnew


# speedup vs baseline: 3.2818x; 1.0005x over previous
import jax
import jax.numpy as jnp
from jax.experimental import pallas as pl

_EDGE_LIST = [(0, 9), (0, 10), (0, 5), (9, 7), (9, 8), (9, 4), (2, 10), (2, 5),
              (7, 9), (7, 5), (7, 4), (8, 9), (8, 4), (6, 0), (6, 2), (6, 5),
              (6, 9), (3, 10), (3, 5), (10, 5), (1, 0), (1, 3)]
_N = 11
_SRC = tuple(e[0] for e in _EDGE_LIST) + tuple(range(_N))
_DST = tuple(e[1] for e in _EDGE_LIST) + tuple(range(_N))
_E = len(_SRC)
_IN_EDGES = tuple(tuple(k for k in range(_E) if _DST[k] == i) for i in range(_N))

_H, _C = 2, 32
_HC = _H * _C
_HID = 32


def _elu(v):
    return jnp.where(v > 0, v, jnp.exp(jnp.minimum(v, 0.0)) - 1.0)


def _gat(x2, Wl, bl, Wr, br, Ab, bias, bb):
    xl = (jnp.dot(x2, Wl, preferred_element_type=jnp.float32) + bl).reshape(_N, bb, _HC)
    xr = (jnp.dot(x2, Wr, preferred_element_type=jnp.float32) + br).reshape(_N, bb, _HC)
    xj = jnp.stack([xl[s] for s in _SRC])
    xi = jnp.stack([xr[d] for d in _DST])
    e = jax.nn.leaky_relu(xj + xi, negative_slope=0.2)
    lb = jnp.dot(e.reshape(_E * bb, _HC), Ab,
                 preferred_element_type=jnp.float32).reshape(_E, bb, _HC)
    ex = jnp.exp(lb)
    m = ex * xj
    rows = []
    for i in range(_N):
        ks = _IN_EDGES[i]
        sm = ex[ks[0]]
        acc = m[ks[0]]
        for k in ks[1:]:
            sm = sm + ex[k]
            acc = acc + m[k]
        rows.append(acc / (sm + 1e-16))
    return jnp.stack(rows) + bias


def _body(x_ref, h0_ref, Wl1, bl1, Wr1, br1, Ab1, b1, Wl2, bl2, Wr2, br2,
          Ab2, b2, WihT, bih, WhhT, bhh, Hw1, Hb1, Hw2, Hb2,
          out_ref, hnew_ref):
    bb = x_ref.shape[0]
    d = x_ref.shape[1] // _N
    xw = x_ref[...]
    x2 = jnp.concatenate([xw[:, n * d:(n + 1) * d] for n in range(_N)], axis=0)
    h = _gat(x2, Wl1[...], bl1[...], Wr1[...], br1[...], Ab1[...], b1[...], bb)
    h = _elu(h)
    h = _gat(h.reshape(_N * bb, _HC), Wl2[...], bl2[...], Wr2[...], br2[...],
             Ab2[...], b2[...], bb)
    h = _elu(h)

    gx2 = h.reshape(_N * bb, _HC)
    h0w = h0_ref[...]
    h02 = jnp.concatenate(
        [h0w[:, n * _HID:(n + 1) * _HID] for n in range(_N)], axis=0)
    gi = jnp.dot(gx2, WihT[...], preferred_element_type=jnp.float32) + bih[...]
    gh = jnp.dot(h02, WhhT[...], preferred_element_type=jnp.float32) + bhh[...]
    rz = jax.nn.sigmoid(gi[:, :2 * _HID] + gh[:, :2 * _HID])
    r = rz[:, :_HID]
    z = rz[:, _HID:]
    n = jnp.tanh(gi[:, 2 * _HID:] + r * gh[:, 2 * _HID:])
    hnew2 = (1.0 - z) * n + z * h02
    t = hnew2.reshape(_N, bb, _HID)
    hnew_ref[...] = jnp.concatenate([t[i] for i in range(_N)], axis=1)
    outs = []
    for i in range(_N):
        h1 = jax.nn.relu(jnp.dot(t[i], Hw1[i], preferred_element_type=jnp.float32)
                         + Hb1[i])
        outs.append(jnp.dot(h1, Hw2[i], preferred_element_type=jnp.float32) + Hb2[i])
    o = jnp.concatenate(outs, axis=1)
    ot = jnp.tanh(o)
    osig = jax.nn.sigmoid(o)
    lane = jax.lax.broadcasted_iota(jnp.int32, o.shape, 1) % 3
    out_ref[...] = jnp.where(lane == 2, osig,
                             jnp.where(lane == 0, ot * 0.3, ot * 0.2))


def _att_mat(att):
    z = jnp.zeros((_C, _C), jnp.float32)
    blocks = []
    for h in range(_H):
        row = [z] * _H
        row[h] = jnp.broadcast_to(att[h][:, None], (_C, _C))
        blocks.append(jnp.concatenate(row, axis=1))
    return jnp.concatenate(blocks, axis=0)


def kernel(x, hidden_state, edge_index, params):
    B, N, D = x.shape
    p = params
    bb = 1024
    xw = x.reshape(B, N * D)
    h0w = hidden_state.reshape(B, N * _HID)

    r2 = lambda v: v.reshape(1, -1)
    weights = [
        p['Wl1'], r2(p['bl1']), p['Wr1'], r2(p['br1']), _att_mat(p['att1']), r2(p['bias1']),
        p['Wl2'], r2(p['bl2']), p['Wr2'], r2(p['br2']), _att_mat(p['att2']), r2(p['bias2']),
        p['Wih'].T, r2(p['bih']), p['Whh'].T, r2(p['bhh']),
        p['Hw1'], p['Hb1'].reshape(N, 1, 16), p['Hw2'], p['Hb2'].reshape(N, 1, 3),
    ]

    grid = (B // bb,)
    batch_spec = lambda f: pl.BlockSpec((bb, f), lambda i: (i, 0))
    w_specs = [pl.BlockSpec(w.shape, (lambda nd: (lambda i: (0,) * nd))(w.ndim))
               for w in weights]

    out_w, hnew_w = pl.pallas_call(
        _body,
        grid=grid,
        in_specs=[batch_spec(N * D), batch_spec(N * _HID)] + w_specs,
        out_specs=[batch_spec(N * 3), batch_spec(N * _HID)],
        out_shape=[jax.ShapeDtypeStruct((B, N * 3), jnp.float32),
                   jax.ShapeDtypeStruct((B, N * _HID), jnp.float32)],
    )(xw, h0w, *weights)

    out = out_w.reshape(B, N, 3)
    hnew = hnew_w.reshape(1, B * N, _HID)
    return out, hnew
